# baseline (speedup 1.0000x reference)
import functools

import jax
import jax.numpy as jnp
from jax import lax
from jax.experimental import pallas as pl
from jax.experimental.pallas import tpu as pltpu
from jax.experimental.pallas import tpu_sc as plsc

_B, _D = 16384, 64
_L = 16

_mesh = plsc.VectorSubcoreMesh(core_axis_name="c", subcore_axis_name="s")


@functools.partial(
    pl.kernel,
    out_type=jax.ShapeDtypeStruct((_L,), jnp.float32),
    mesh=_mesh,
    compiler_params=pltpu.CompilerParams(needs_layout_passes=False),
    scratch_types=[
        pltpu.VMEM((_L,), jnp.float32),
    ],
)
def _sc_probe(gu_h, out_h, buf):
    wid = lax.axis_index("s") * 2 + lax.axis_index("c")

    @pl.when(wid == 0)
    def _():
        pltpu.sync_copy(gu_h.at[pl.ds(0, _L)], buf)
        pltpu.sync_copy(buf, out_h)


def _tc_body(a_ref, b_ref, c_ref, d_ref, o_ref):
    o_ref[:] = jnp.sum(a_ref[:] * b_ref[:] + c_ref[:] * d_ref[:], axis=1)


def kernel(gu, gi, gut, git):
    probe = _sc_probe(gu.reshape(-1))
    BLK = 2048
    out = pl.pallas_call(
        _tc_body,
        grid=(_B // BLK,),
        in_specs=[pl.BlockSpec((BLK, _D), lambda i: (i, 0))] * 4,
        out_specs=pl.BlockSpec((BLK,), lambda i: (i,)),
        out_shape=jax.ShapeDtypeStruct((_B,), jnp.float32),
    )(gu, gi, gut, git)
    return out + 0.0 * probe[0]
